# baseline (speedup 1.0000x reference)
import functools
import jax
import jax.numpy as jnp
import numpy as np
from jax.experimental import pallas as pl
from jax.experimental.pallas import tpu as pltpu

N = 10000
D = 128
NUM_L = 1500
NUM_H = 9500
NSEL = NUM_H - NUM_L
LAM = 0.35

_f32 = jnp.float32


def _fts_body(s1, s2, w1, w2, f1, f2, f3, f4):
    f1[...] = jnp.dot(s1[...], w1[...], preferred_element_type=_f32)
    f2[...] = jnp.dot(s1[...], w2[...], preferred_element_type=_f32)
    f3[...] = jnp.dot(s2[...], w1[...], preferred_element_type=_f32)
    f4[...] = jnp.dot(s2[...], w2[...], preferred_element_type=_f32)


def _fts(s1, s2, w1, w2, bm=1000):
    g = N // bm
    spec = pl.BlockSpec((bm, D), lambda i: (i, 0))
    wspec = pl.BlockSpec((D, D), lambda i: (0, 0))
    return pl.pallas_call(
        _fts_body,
        grid=(g,),
        in_specs=[spec, spec, wspec, wspec],
        out_specs=[spec] * 4,
        out_shape=[jax.ShapeDtypeStruct((N, D), _f32)] * 4,
        compiler_params=pltpu.CompilerParams(
            dimension_semantics=("parallel",)),
    )(s1, s2, w1, w2)


def _gcn_body(a_ref, fa_ref, fb_ref, ba_ref, bb_ref, al_ref, bl_ref,
              ha_ref, hb_ref):
    TS = 256
    nt = (N + TS - 1) // TS

    def tdot(f_ref, k):
        k0 = k * TS
        k1 = min(k0 + TS, N)
        return jnp.dot(a_ref[:, k0:k1], f_ref[k0:k1, :],
                       preferred_element_type=_f32)

    acca = tdot(fa_ref, 0)
    accb = tdot(fb_ref, 0)
    for k in range(1, nt):
        acca = acca + tdot(fa_ref, k)
        accb = accb + tdot(fb_ref, k)
    oa = acca + ba_ref[...]
    ob = accb + bb_ref[...]
    ha_ref[...] = jnp.where(oa >= 0, oa, al_ref[0, 0] * oa)
    hb_ref[...] = jnp.where(ob >= 0, ob, bl_ref[0, 0] * ob)


def _gcn(a, fa, fb, ba, bb, alpha_a, alpha_b, bm=400):
    g = N // bm
    return pl.pallas_call(
        _gcn_body,
        grid=(g,),
        in_specs=[
            pl.BlockSpec((bm, N), lambda i: (i, 0)),
            pl.BlockSpec((N, D), lambda i: (0, 0)),
            pl.BlockSpec((N, D), lambda i: (0, 0)),
            pl.BlockSpec((1, D), lambda i: (0, 0)),
            pl.BlockSpec((1, D), lambda i: (0, 0)),
            pl.BlockSpec((1, 1), lambda i: (0, 0)),
            pl.BlockSpec((1, 1), lambda i: (0, 0)),
        ],
        out_specs=[pl.BlockSpec((bm, D), lambda i: (i, 0))] * 2,
        out_shape=[jax.ShapeDtypeStruct((N, D), _f32)] * 2,
        compiler_params=pltpu.CompilerParams(
            dimension_semantics=("parallel",),
            vmem_limit_bytes=100 * 1024 * 1024),
    )(a, fa, fb, ba, bb, alpha_a, alpha_b)


def _norms_body(h1, h2, h3, h4, n1, n2, n3, n4):
    for h, n in ((h1, n1), (h2, n2), (h3, n3), (h4, n4)):
        x = h[...]
        y = (x * x).reshape(-1, 16, 8)
        p = y[:, 0, :]
        for v in range(1, 16):
            p = p + y[:, v, :]
        a4 = p[:, 0:4] + p[:, 4:8]
        a2 = a4[:, 0:2] + a4[:, 2:4]
        n[...] = jnp.sqrt(a2[:, 0:1] + a2[:, 1:2])


def _norms(h1, h2, h3, h4, bm=1000):
    g = N // bm
    hspec = pl.BlockSpec((bm, D), lambda i: (i, 0))
    nspec = pl.BlockSpec((bm, 1), lambda i: (i, 0))
    return pl.pallas_call(
        _norms_body,
        grid=(g,),
        in_specs=[hspec] * 4,
        out_specs=[nspec] * 4,
        out_shape=[jax.ShapeDtypeStruct((N, 1), _f32)] * 4,
        compiler_params=pltpu.CompilerParams(
            dimension_semantics=("parallel",),
            vmem_limit_bytes=100 * 1024 * 1024),
    )(h1, h2, h3, h4)


def _small_body(h1, h2, msk, wd, vpack):
    m = msk[...]
    msum = jnp.sum(m)
    s1 = jnp.sum(h1[...] * m, axis=0, keepdims=True) / msum
    s2 = jnp.sum(h2[...] * m, axis=0, keepdims=True) / msum
    c1 = jax.nn.sigmoid(s1)
    c2 = jax.nn.sigmoid(s2)
    w = wd[...]
    v1 = jax.lax.dot_general(c1, w, (((1,), (1,)), ((), ())),
                             preferred_element_type=_f32)
    v2 = jax.lax.dot_general(c2, w, (((1,), (1,)), ((), ())),
                             preferred_element_type=_f32)
    col = jax.lax.broadcasted_iota(jnp.int32, (1, D), 1)
    mix = (col % 6 == 0).astype(_f32)
    vpack[...] = jnp.concatenate(
        [v1, v2,
         v1 * (1.0 - LAM * mix), v1 * (LAM * mix),
         v2 * (1.0 - LAM * mix), v2 * (LAM * mix),
         v1, v2], axis=0)


def _small(h1, h2, msk, wd):
    full = pl.BlockSpec((N, D), lambda: (0, 0))
    col = pl.BlockSpec((N, 1), lambda: (0, 0))
    return pl.pallas_call(
        _small_body,
        in_specs=[full, full, col,
                  pl.BlockSpec((D, D), lambda: (0, 0))],
        out_specs=pl.BlockSpec((8, D), lambda: (0, 0)),
        out_shape=jax.ShapeDtypeStruct((8, D), _f32),
        compiler_params=pltpu.CompilerParams(
            vmem_limit_bytes=100 * 1024 * 1024),
    )(h1, h2, msk, wd)


def _sim_body(hx_ref, hy_ref, nx_ref, ny_ref, out_ref):
    j = pl.program_id(1)
    x = jax.lax.dot_general(hx_ref[...], hy_ref[...], (((1,), (1,)), ((), ())),
                            preferred_element_type=_f32)
    t = x / (nx_ref[...] * jnp.transpose(ny_ref[...]))
    m = jnp.max(t, axis=1, keepdims=True)

    @pl.when(j == 0)
    def _():
        out_ref[...] = m

    @pl.when(j > 0)
    def _():
        out_ref[...] = jnp.maximum(out_ref[...], m)


def _sim(hx, hy, nx, ny, bm=1000, bn=2000):
    gi, gj = N // bm, N // bn
    return pl.pallas_call(
        _sim_body,
        grid=(gi, gj),
        in_specs=[
            pl.BlockSpec((bm, D), lambda i, j: (i, 0)),
            pl.BlockSpec((bn, D), lambda i, j: (j, 0)),
            pl.BlockSpec((bm, 1), lambda i, j: (i, 0)),
            pl.BlockSpec((bn, 1), lambda i, j: (j, 0)),
        ],
        out_specs=pl.BlockSpec((bm, 1), lambda i, j: (i, 0)),
        out_shape=jax.ShapeDtypeStruct((N, 1), _f32),
        compiler_params=pltpu.CompilerParams(
            dimension_semantics=("parallel", "arbitrary"),
            vmem_limit_bytes=100 * 1024 * 1024),
    )(hx, hy, nx, ny)


def _rank_body(si1, sj1, si2, sj2, r1, r2, *, bi, bj):
    i, j = pl.program_id(0), pl.program_id(1)
    ii = jax.lax.broadcasted_iota(jnp.int32, (bi, 1), 0) + i * bi
    jj = jax.lax.broadcasted_iota(jnp.int32, (1, bj), 1) + j * bj

    def count(si_ref, sj_ref):
        si = si_ref[...]
        sj = jnp.transpose(sj_ref[...])
        lt = (sj < si) | ((sj == si) & (jj < ii))
        return jnp.sum(lt.astype(jnp.int32), axis=1, keepdims=True)

    c1 = count(si1, sj1)
    c2 = count(si2, sj2)

    @pl.when(j == 0)
    def _():
        r1[...] = c1
        r2[...] = c2

    @pl.when(j > 0)
    def _():
        r1[...] = r1[...] + c1
        r2[...] = r2[...] + c2


def _rank(s1, s2, bi=1000, bj=2000):
    gi, gj = N // bi, N // bj
    ispec = pl.BlockSpec((bi, 1), lambda i, j: (i, 0))
    jspec = pl.BlockSpec((bj, 1), lambda i, j: (j, 0))
    return pl.pallas_call(
        functools.partial(_rank_body, bi=bi, bj=bj),
        grid=(gi, gj),
        in_specs=[ispec, jspec, ispec, jspec],
        out_specs=[ispec, ispec],
        out_shape=[jax.ShapeDtypeStruct((N, 1), jnp.int32)] * 2,
        compiler_params=pltpu.CompilerParams(
            dimension_semantics=("parallel", "arbitrary"),
            vmem_limit_bytes=100 * 1024 * 1024),
    )(s1, s1, s2, s2)


def _scores_body(h1, h2, h3, h4, vp, sb1, sb2, bd_ref,
                 sc1, sc2, u3, w3, u4, w4):
    v = vp[...]

    def mv(h_ref, row):
        vr = v[row:row + 1, :]
        return jnp.sum(h_ref[...] * vr, axis=1, keepdims=True)

    bd = bd_ref[0, 0]
    sc1[...] = mv(h2, 0) + bd + sb1[...]
    sc2[...] = mv(h1, 1) + bd + sb2[...]
    u4[...] = mv(h4, 2)
    w4[...] = mv(h4, 3)
    u3[...] = mv(h3, 4)
    w3[...] = mv(h3, 5)


def _scores(h1, h2, h3, h4, vpack, sb1, sb2, bd, bm=1000):
    g = N // bm
    hspec = pl.BlockSpec((bm, D), lambda i: (i, 0))
    cspec = pl.BlockSpec((bm, 1), lambda i: (i, 0))
    return pl.pallas_call(
        _scores_body,
        grid=(g,),
        in_specs=[hspec, hspec, hspec, hspec,
                  pl.BlockSpec((8, D), lambda i: (0, 0)),
                  cspec, cspec,
                  pl.BlockSpec((1, 1), lambda i: (0, 0))],
        out_specs=[cspec] * 6,
        out_shape=[jax.ShapeDtypeStruct((N, 1), _f32)] * 6,
        compiler_params=pltpu.CompilerParams(
            dimension_semantics=("parallel",)),
    )(h1, h2, h3, h4, vpack, sb1, sb2, bd)


def _scatter_body(r1, r2, u3, w3, u4, w4, idxp, bd_ref, sc3, sc4, *, bp, bi):
    p = pl.program_id(0)
    i = pl.program_id(1)
    pcol = jax.lax.broadcasted_iota(jnp.int32, (bp, 1), 0) + (
        p * bp + NUM_L)
    pidx = idxp[...] + NUM_L

    def onehot_dot(r_ref, a_ref, b_ref):
        rrow = jnp.transpose(r_ref[...])
        oha = (rrow == pcol).astype(_f32)
        ohb = (rrow == pidx).astype(_f32)
        return (jnp.dot(oha, a_ref[...], preferred_element_type=_f32)
                + jnp.dot(ohb, b_ref[...], preferred_element_type=_f32))

    c3 = onehot_dot(r2, u4, w4)
    c4 = onehot_dot(r1, u3, w3)

    @pl.when(i == 0)
    def _():
        sc3[...] = c3 + bd_ref[0, 0]
        sc4[...] = c4 + bd_ref[0, 0]

    @pl.when(i > 0)
    def _():
        sc3[...] = sc3[...] + c3
        sc4[...] = sc4[...] + c4


def _scatter(r1, r2, u3, w3, u4, w4, idxp, bd, bp=400, bi=2000):
    gp, gi = NSEL // bp, N // bi
    pspec = pl.BlockSpec((bp, 1), lambda p, i: (p, 0))
    ispec = pl.BlockSpec((bi, 1), lambda p, i: (i, 0))
    return pl.pallas_call(
        functools.partial(_scatter_body, bp=bp, bi=bi),
        grid=(gp, gi),
        in_specs=[ispec, ispec, ispec, ispec, ispec, ispec, pspec,
                  pl.BlockSpec((1, 1), lambda p, i: (0, 0))],
        out_specs=[pspec, pspec],
        out_shape=[jax.ShapeDtypeStruct((NSEL, 1), _f32)] * 2,
        compiler_params=pltpu.CompilerParams(
            dimension_semantics=("parallel", "arbitrary")),
    )(r1, r2, u3, w3, u4, w4, idxp, bd)


def kernel(seq1, seq2, adj, diff, sparse, msk, samp_bias1, samp_bias2,
           epoch, W1, b1, a1, W2, b2, a2, Wd, bd):
    s1 = seq1[0]
    s2 = seq2[0]
    A = adj[0]
    Df = diff[0]
    b1r = b1.reshape(1, D)
    b2r = b2.reshape(1, D)
    a1r = a1.reshape(1, 1)
    a2r = a2.reshape(1, 1)
    bdr = bd.reshape(1, 1)
    mskc = msk.reshape(N, 1)
    sb1 = samp_bias1.reshape(N, 1)
    sb2 = samp_bias2.reshape(N, 1)

    f1, f2, f3, f4 = _fts(s1, s2, W1, W2)
    h1, h3 = _gcn(A, f1, f3, b1r, b1r, a1r, a1r)
    h2, h4 = _gcn(Df, f2, f4, b2r, b2r, a2r, a2r)
    n1, n2, n3, n4 = _norms(h1, h2, h3, h4)
    vpack = _small(h1, h2, mskc, Wd)
    sim1 = _sim(h3, h1, n3, n1)
    sim2 = _sim(h4, h2, n4, n2)
    r1, r2 = _rank(sim1, sim2)
    sc1, sc2, u3, w3, u4, w4 = _scores(h1, h2, h3, h4, vpack, sb1, sb2, bdr)

    idx = jax.random.permutation(jax.random.key(42), NSEL)
    idxp = jnp.asarray(idx, jnp.int32).reshape(NSEL, 1)
    sc3, sc4 = _scatter(r1, r2, u3, w3, u4, w4, idxp, bdr)

    return jnp.concatenate(
        [sc1.reshape(1, N), sc2.reshape(1, N),
         sc3.reshape(1, NSEL), sc4.reshape(1, NSEL)], axis=1)
